# baseline (speedup 1.0000x reference)
import jax
import jax.numpy as jnp
from jax.experimental import pallas as pl
from jax.experimental.pallas import tpu as pltpu
from jax.experimental.pallas import tpu_sc as plsc

_B = 16
_N = 8192
_K_NOR = 5
_K_ABN = 10
_K_HARD = 10
_K_TOT = _K_NOR + _K_ABN + _K_HARD
_W = 16
_N_GATHER = 512


def _topk_body(s_ref, idx_ref, val_ref, frow_ref):
    s = s_ref[...]
    iota = jax.lax.broadcasted_iota(jnp.int32, s.shape, 1)
    neg = jnp.float32(-3.0e38)

    def take_topk(z, k, collect_from=None):
        idx_cols, val_cols = [], []
        for _ in range(k):
            m = jnp.max(z, axis=1, keepdims=True)
            sel_any = z == m
            idx = jnp.min(jnp.where(sel_any, iota, jnp.int32(_N)),
                          axis=1, keepdims=True)
            sel = iota == idx
            if collect_from is None:
                v = m
            else:
                v = jnp.sum(jnp.where(sel, collect_from, jnp.float32(0.0)),
                            axis=1, keepdims=True)
            z = jnp.where(sel, neg, z)
            idx_cols.append(idx)
            val_cols.append(v)
        return idx_cols, val_cols

    i_nor, v_nor = take_topk(-s, _K_NOR)
    i_abn, v_abn = take_topk(s, _K_ABN)
    i_hard, v_hard = take_topk(-jnp.abs(s - 0.5), _K_HARD, collect_from=s)
    v_nor = [-v for v in v_nor]

    idx_all = jnp.concatenate(i_nor + i_abn + i_hard, axis=1)
    val_all = jnp.concatenate(v_nor + v_abn + v_hard, axis=1)
    pad_i = jnp.zeros((_B, 128 - _K_TOT), jnp.int32)
    pad_v = jnp.zeros((_B, 128 - _K_TOT), jnp.float32)
    idx_ref[...] = jnp.concatenate([idx_all, pad_i], axis=1)
    val_ref[...] = jnp.concatenate([val_all, pad_v], axis=1)

    base = jax.lax.broadcasted_iota(jnp.int32, (_B, 1), 0) * _N
    zc = jnp.zeros((_B, 1), jnp.int32)
    row_a = jnp.concatenate(
        [c + base for c in (i_nor + i_abn)] + [zc] * (128 - 15), axis=1)
    row_b = jnp.concatenate(
        [c + base for c in i_hard] + [zc] * (128 - 10), axis=1)
    frow_ref[...] = jnp.concatenate([row_a, row_b], axis=0)


def _run_topk(score):
    return pl.pallas_call(
        _topk_body,
        out_shape=[jax.ShapeDtypeStruct((_B, 128), jnp.int32),
                   jax.ShapeDtypeStruct((_B, 128), jnp.float32),
                   jax.ShapeDtypeStruct((32, 128), jnp.int32)],
    )(score)


def _sc_gather(feat2d, idx_rows):
    f_dim = feat2d.shape[1]
    mesh = plsc.VectorSubcoreMesh(core_axis_name="core",
                                  subcore_axis_name="subcore",
                                  num_cores=2, num_subcores=16)

    @pl.kernel(out_type=jax.ShapeDtypeStruct((_N_GATHER, f_dim),
                                             feat2d.dtype),
               mesh=mesh,
               scratch_types=[pltpu.VMEM((128,), jnp.int32),
                              pltpu.VMEM((128,), jnp.int32),
                              pltpu.VMEM((8, 1024), jnp.float32),
                              pltpu.VMEM((8, 1024), jnp.float32),
                              pltpu.VMEM((8, 1024), jnp.float32),
                              pltpu.VMEM((8, 1024), jnp.float32),
                              pltpu.SemaphoreType.DMA,
                              pltpu.SemaphoreType.DMA,
                              pltpu.SemaphoreType.DMA,
                              pltpu.SemaphoreType.DMA,
                              pltpu.SemaphoreType.DMA,
                              pltpu.SemaphoreType.DMA])
    def knl(x_hbm, i_hbm, o_hbm, idx_a, idx_b, buf_a, buf_b, buf_c,
            buf_d, semi1, semi2, sem1, sem2, sem3, sem4):
        c = jax.lax.axis_index("core")
        s = jax.lax.axis_index("subcore")
        sid = c * 16 + s
        pltpu.async_copy(i_hbm.at[sid], idx_a, semi1).wait()
        cg1 = pltpu.async_copy(x_hbm.at[idx_a.at[pl.ds(0, 8)]], buf_a,
                               sem1)
        cg2 = pltpu.async_copy(x_hbm.at[idx_a.at[pl.ds(8, 8)]], buf_b,
                               sem2)
        cg1.wait()
        cp1 = pltpu.async_copy(buf_a, o_hbm.at[pl.ds(sid * _W, 8)], sem1)
        cg2.wait()
        cp2 = pltpu.async_copy(buf_b, o_hbm.at[pl.ds(sid * _W + 8, 8)],
                               sem2)
        cp1.wait()
        cp2.wait()

    return knl(feat2d, idx_rows)


def _unpack_body(g_ref, i_ref, v_ref,
                 fn_ref, sn_ref, in_ref, fa_ref, sa_ref,
                 fh_ref, sh_ref, ih_ref):
    gv = g_ref[...]
    g1 = gv[:_B * _W].reshape(_B, _W, 1024)
    g2 = gv[_B * _W:].reshape(_B, _W, 1024)
    fn_ref[...] = g1[:, 0:_K_NOR]
    fa_ref[...] = g1[:, _K_NOR:_K_NOR + _K_ABN]
    fh_ref[...] = g2[:, 0:_K_HARD]
    iv = i_ref[...]
    vv = v_ref[...]
    in_ref[...] = iv[:, 0:_K_NOR]
    ih_ref[...] = iv[:, _K_NOR + _K_ABN:_K_TOT]
    sn_ref[...] = vv[:, 0:_K_NOR]
    sa_ref[...] = vv[:, _K_NOR:_K_NOR + _K_ABN]
    sh_ref[...] = vv[:, _K_NOR + _K_ABN:_K_TOT]


def _run_unpack(g, out_i, out_v, f_dim):
    f32, i32 = jnp.float32, jnp.int32
    return pl.pallas_call(
        _unpack_body,
        out_shape=[jax.ShapeDtypeStruct((_B, _K_NOR, f_dim), f32),
                   jax.ShapeDtypeStruct((_B, _K_NOR), f32),
                   jax.ShapeDtypeStruct((_B, _K_NOR), i32),
                   jax.ShapeDtypeStruct((_B, _K_ABN, f_dim), f32),
                   jax.ShapeDtypeStruct((_B, _K_ABN), f32),
                   jax.ShapeDtypeStruct((_B, _K_HARD, f_dim), f32),
                   jax.ShapeDtypeStruct((_B, _K_HARD), f32),
                   jax.ShapeDtypeStruct((_B, _K_HARD), i32)],
    )(g, out_i, out_v)


def kernel(feat, score):
    b, n, f_dim = feat.shape
    out_i, out_v, idx_rows = _run_topk(score)

    idx_nor = out_i[:, 0:_K_NOR]
    val_nor = out_v[:, 0:_K_NOR]
    val_abn = out_v[:, _K_NOR:_K_NOR + _K_ABN]
    idx_hard = out_i[:, _K_NOR + _K_ABN:_K_TOT]
    val_hard = out_v[:, _K_NOR + _K_ABN:_K_TOT]

    g = _sc_gather(feat.reshape(b * n, f_dim), idx_rows)
    g1 = g[:b * _W].reshape(b, _W, f_dim)
    g2 = g[b * _W:].reshape(b, _W, f_dim)

    feat_nor = g1[:, 0:_K_NOR]
    feat_abn = g1[:, _K_NOR:_K_NOR + _K_ABN]
    feat_hard = g2[:, 0:_K_HARD]

    return (feat_nor, val_nor, idx_nor,
            feat_abn, val_abn,
            feat_hard, val_hard, idx_hard)


# speedup vs baseline: 1.0115x; 1.0115x over previous
import jax
import jax.numpy as jnp
from jax.experimental import pallas as pl
from jax.experimental.pallas import tpu as pltpu
from jax.experimental.pallas import tpu_sc as plsc

_B = 16
_N = 8192
_K_NOR = 5
_K_ABN = 10
_K_HARD = 10
_K_TOT = _K_NOR + _K_ABN + _K_HARD
_W = 16
_N_GATHER = 512


def _topk_body(s_ref, idx_ref, val_ref, frow_ref):
    s = s_ref[...]
    iota = jax.lax.broadcasted_iota(jnp.int32, s.shape, 1)
    neg = jnp.float32(-3.0e38)

    def take_topk(z, k, collect_from=None):
        idx_cols, val_cols = [], []
        for _ in range(k):
            m = jnp.max(z, axis=1, keepdims=True)
            sel_any = z == m
            idx = jnp.min(jnp.where(sel_any, iota, jnp.int32(_N)),
                          axis=1, keepdims=True)
            sel = iota == idx
            if collect_from is None:
                v = m
            else:
                v = jnp.sum(jnp.where(sel, collect_from, jnp.float32(0.0)),
                            axis=1, keepdims=True)
            z = jnp.where(sel, neg, z)
            idx_cols.append(idx)
            val_cols.append(v)
        return idx_cols, val_cols

    i_nor, v_nor = take_topk(-s, _K_NOR)
    i_abn, v_abn = take_topk(s, _K_ABN)
    i_hard, v_hard = take_topk(-jnp.abs(s - 0.5), _K_HARD, collect_from=s)
    v_nor = [-v for v in v_nor]

    idx_all = jnp.concatenate(i_nor + i_abn + i_hard, axis=1)
    val_all = jnp.concatenate(v_nor + v_abn + v_hard, axis=1)
    pad_i = jnp.zeros((_B, 128 - _K_TOT), jnp.int32)
    pad_v = jnp.zeros((_B, 128 - _K_TOT), jnp.float32)
    idx_ref[...] = jnp.concatenate([idx_all, pad_i], axis=1)
    val_ref[...] = jnp.concatenate([val_all, pad_v], axis=1)

    base = jax.lax.broadcasted_iota(jnp.int32, (_B, 1), 0) * _N
    zc = jnp.zeros((_B, 1), jnp.int32)
    row_a = jnp.concatenate(
        [c + base for c in (i_nor + i_abn)] + [zc] * (128 - 15), axis=1)
    row_b = jnp.concatenate(
        [c + base for c in i_hard] + [zc] * (128 - 10), axis=1)
    frow_ref[...] = jnp.concatenate([row_a, row_b], axis=0)


def _run_topk(score):
    return pl.pallas_call(
        _topk_body,
        out_shape=[jax.ShapeDtypeStruct((_B, 128), jnp.int32),
                   jax.ShapeDtypeStruct((_B, 128), jnp.float32),
                   jax.ShapeDtypeStruct((32, 128), jnp.int32)],
    )(score)


def _sc_gather(feat2d, idx_rows):
    f_dim = feat2d.shape[1]
    mesh = plsc.VectorSubcoreMesh(core_axis_name="core",
                                  subcore_axis_name="subcore",
                                  num_cores=1, num_subcores=16)

    @pl.kernel(out_type=jax.ShapeDtypeStruct((_N_GATHER, f_dim),
                                             feat2d.dtype),
               mesh=mesh,
               scratch_types=[pltpu.VMEM((128,), jnp.int32),
                              pltpu.VMEM((128,), jnp.int32),
                              pltpu.VMEM((8, 1024), jnp.float32),
                              pltpu.VMEM((8, 1024), jnp.float32),
                              pltpu.VMEM((8, 1024), jnp.float32),
                              pltpu.VMEM((8, 1024), jnp.float32),
                              pltpu.SemaphoreType.DMA,
                              pltpu.SemaphoreType.DMA,
                              pltpu.SemaphoreType.DMA,
                              pltpu.SemaphoreType.DMA,
                              pltpu.SemaphoreType.DMA,
                              pltpu.SemaphoreType.DMA])
    def knl(x_hbm, i_hbm, o_hbm, idx_a, idx_b, buf_a, buf_b, buf_c,
            buf_d, semi1, semi2, sem1, sem2, sem3, sem4):
        s = jax.lax.axis_index("subcore")
        r0 = 2 * s
        r1 = 2 * s + 1
        cpi_a = pltpu.async_copy(i_hbm.at[r0], idx_a, semi1)
        cpi_b = pltpu.async_copy(i_hbm.at[r1], idx_b, semi2)
        cpi_a.wait()
        cg1 = pltpu.async_copy(x_hbm.at[idx_a.at[pl.ds(0, 8)]], buf_a,
                               sem1)
        cg2 = pltpu.async_copy(x_hbm.at[idx_a.at[pl.ds(8, 8)]], buf_b,
                               sem2)
        cpi_b.wait()
        cg3 = pltpu.async_copy(x_hbm.at[idx_b.at[pl.ds(0, 8)]], buf_c,
                               sem3)
        cg4 = pltpu.async_copy(x_hbm.at[idx_b.at[pl.ds(8, 8)]], buf_d,
                               sem4)
        cg1.wait()
        cp1 = pltpu.async_copy(buf_a, o_hbm.at[pl.ds(r0 * _W, 8)], sem1)
        cg2.wait()
        cp2 = pltpu.async_copy(buf_b, o_hbm.at[pl.ds(r0 * _W + 8, 8)],
                               sem2)
        cg3.wait()
        cp3 = pltpu.async_copy(buf_c, o_hbm.at[pl.ds(r1 * _W, 8)], sem3)
        cg4.wait()
        cp4 = pltpu.async_copy(buf_d, o_hbm.at[pl.ds(r1 * _W + 8, 8)],
                               sem4)
        cp1.wait()
        cp2.wait()
        cp3.wait()
        cp4.wait()

    return knl(feat2d, idx_rows)


def kernel(feat, score):
    b, n, f_dim = feat.shape
    out_i, out_v, idx_rows = _run_topk(score)

    idx_nor = out_i[:, 0:_K_NOR]
    val_nor = out_v[:, 0:_K_NOR]
    val_abn = out_v[:, _K_NOR:_K_NOR + _K_ABN]
    idx_hard = out_i[:, _K_NOR + _K_ABN:_K_TOT]
    val_hard = out_v[:, _K_NOR + _K_ABN:_K_TOT]

    g = _sc_gather(feat.reshape(b * n, f_dim), idx_rows)
    g1 = g[:b * _W].reshape(b, _W, f_dim)
    g2 = g[b * _W:].reshape(b, _W, f_dim)

    feat_nor = g1[:, 0:_K_NOR]
    feat_abn = g1[:, _K_NOR:_K_NOR + _K_ABN]
    feat_hard = g2[:, 0:_K_HARD]

    return (feat_nor, val_nor, idx_nor,
            feat_abn, val_abn,
            feat_hard, val_hard, idx_hard)
